# baseline (speedup 1.0000x reference)
import functools

import jax
import jax.numpy as jnp
from jax import lax
from jax.experimental import pallas as pl
from jax.experimental.pallas import tpu as pltpu
from jax.experimental.pallas import tpu_sc as plsc

NUM_CORES = 2
NUM_SUBCORES = 16
LANES = 16
NUM_WORKERS = NUM_CORES * NUM_SUBCORES

ROWS_PER_CHUNK = 32


@functools.partial(jax.jit, static_argnums=(2, 3))
def _onehot_sc(zeros_flat, idx, batch, nb_digits):
    rows_per_worker = batch // NUM_WORKERS
    chunks = rows_per_worker // ROWS_PER_CHUNK
    chunk_elems = ROWS_PER_CHUNK * nb_digits

    mesh = plsc.VectorSubcoreMesh(core_axis_name="c", subcore_axis_name="s")

    def body(zeros_hbm, idx_hbm, out_hbm, idx_v, buf0, buf1):
        wid = lax.axis_index("s") * NUM_CORES + lax.axis_index("c")
        row_base = wid * rows_per_worker
        elem_base = row_base * nb_digits

        pltpu.sync_copy(idx_hbm.at[pl.ds(row_base, rows_per_worker)], idx_v)
        pltpu.sync_copy(zeros_hbm.at[pl.ds(0, chunk_elems)], buf0)
        pltpu.sync_copy(zeros_hbm.at[pl.ds(0, chunk_elems)], buf1)

        iota = lax.iota(jnp.int32, LANES)
        ones = jnp.full((LANES,), 1.0, jnp.float32)
        zeros = jnp.zeros((LANES,), jnp.float32)
        bufs = (buf0, buf1)

        def flat_pos(c, j):
            col_v = idx_v[pl.ds(c * ROWS_PER_CHUNK + j * LANES, LANES)]
            return (iota + j * LANES) * nb_digits + col_v

        for c in range(chunks):
            buf = bufs[c % 2]
            for j in range(ROWS_PER_CHUNK // LANES):
                plsc.store_scatter(buf, [flat_pos(c, j)], ones)
            pltpu.sync_copy(buf, out_hbm.at[pl.ds(elem_base + c * chunk_elems,
                                                  chunk_elems)])
            for j in range(ROWS_PER_CHUNK // LANES):
                plsc.store_scatter(buf, [flat_pos(c, j)], zeros)

    f = pl.kernel(
        body,
        out_type=jax.ShapeDtypeStruct((batch * nb_digits,), jnp.float32),
        mesh=mesh,
        scratch_types=[
            pltpu.VMEM((rows_per_worker,), jnp.int32),
            pltpu.VMEM((chunk_elems,), jnp.float32),
            pltpu.VMEM((chunk_elems,), jnp.float32),
        ],
        compiler_params=pltpu.CompilerParams(needs_layout_passes=False),
    )
    return f(zeros_flat, idx)


def kernel(onehot_buf, long_tensor, nb_digits):
    del nb_digits
    batch, digits = onehot_buf.shape
    idx = long_tensor.reshape(-1).astype(jnp.int32)
    flat = _onehot_sc(onehot_buf.reshape(-1), idx, batch, digits)
    return flat.reshape(batch, digits)
